# Initial kernel scaffold; baseline (speedup 1.0000x reference)
import jax
import jax.numpy as jnp
from jax.experimental import pallas as pl


def kernel(x, edge_index, edge_weights, lin_w, lin_b, bn_gamma, bn_beta, fc_w, fc_b):
    raise NotImplementedError("write your pallas kernel here")


# speedup vs baseline: 8.9594x; 8.9594x over previous
import functools
import jax
import jax.numpy as jnp
from jax import lax
from jax.experimental import pallas as pl
from jax.experimental.pallas import tpu as pltpu
from jax.experimental.pallas import tpu_sc as plsc

N = 10000
NPAD = 10240
F_IN = 128
HID = 64
W128 = 128
POOL = 8
OUT = 10
E = 320000
NTILES = 32
NS = 16
CHUNK = 128
NCHUNK = 79
EPT = CHUNK * NCHUNK
EPAD = EPT * NTILES
TRASH = 10200
ROWS_PT = NPAD // NS

_SC_MESH = plsc.VectorSubcoreMesh(core_axis_name="c", subcore_axis_name="s",
                                  num_cores=2, num_subcores=NS)


def _pre_body(x_ref, w_ref, dega_ref, degb_ref, u0_ref, dinv_ref):
    xb = x_ref[...]
    ss = jnp.sum(xb * xb, axis=1, keepdims=True)
    nrm = jnp.sqrt(ss)
    y = xb / jnp.maximum(nrm, 1e-12)
    z = lax.dot_general(y, w_ref[...], (((1,), (1,)), ((), ())),
                        preferred_element_type=jnp.float32)
    deg = dega_ref[...] + degb_ref[...] + 1.0
    dinv = jnp.where(deg > 0, lax.rsqrt(jnp.maximum(deg, 1e-12)), 0.0)
    u0_ref[...] = jnp.concatenate([z * dinv, jnp.zeros_like(z)], axis=1)
    dinv_ref[...] = dinv


def _mid_body(pa_ref, pb_ref, u0_ref, dinv_ref, out_ref):
    dinv = dinv_ref[...]
    u1 = pa_ref[...] + pb_ref[...] - u0_ref[...]
    out_ref[...] = u1 * (dinv * dinv)


def _post_body(qa_ref, qb_ref, u1_ref, dinv_ref, g_ref, linb_ref, gam_ref,
               beta_ref, fcb_ref, out_ref):
    i = pl.program_id(0)
    u2 = qa_ref[...] + qb_ref[...] - u1_ref[...]
    h = u2[:, :HID] * dinv_ref[...] + linb_ref[...]
    e = jnp.where(h > 0, h, jnp.exp(jnp.minimum(h, 0.0)) - 1.0)
    s = gam_ref[0, 0] * lax.rsqrt(jnp.asarray(1.0 + 1e-5, jnp.float32))
    e2 = e * s + beta_ref[0, 0]
    part = jnp.sum(g_ref[...] * e2[None, :, :], axis=(1, 2))

    @pl.when(i == 0)
    def _():
        out_ref[...] = fcb_ref[...]

    out_ref[...] += part[None, :]


def _tc_pre(xp, lin_w, dega, degb):
    blk = 1024
    grid = NPAD // blk
    return pl.pallas_call(
        _pre_body,
        grid=(grid,),
        in_specs=[
            pl.BlockSpec((blk, F_IN), lambda i: (i, 0)),
            pl.BlockSpec((HID, F_IN), lambda i: (0, 0)),
            pl.BlockSpec((blk, 1), lambda i: (i, 0)),
            pl.BlockSpec((blk, 1), lambda i: (i, 0)),
        ],
        out_specs=[
            pl.BlockSpec((blk, W128), lambda i: (i, 0)),
            pl.BlockSpec((blk, 1), lambda i: (i, 0)),
        ],
        out_shape=[
            jax.ShapeDtypeStruct((NPAD, W128), jnp.float32),
            jax.ShapeDtypeStruct((NPAD, 1), jnp.float32),
        ],
    )(xp, lin_w, dega, degb)


def _tc_mid(pa, pb, u0, dinv):
    blk = 1024
    grid = NPAD // blk
    return pl.pallas_call(
        _mid_body,
        grid=(grid,),
        in_specs=[pl.BlockSpec((blk, W128), lambda i: (i, 0))] * 3
        + [pl.BlockSpec((blk, 1), lambda i: (i, 0))],
        out_specs=pl.BlockSpec((blk, W128), lambda i: (i, 0)),
        out_shape=jax.ShapeDtypeStruct((NPAD, W128), jnp.float32),
    )(pa, pb, u0, dinv)


def _tc_post(qa, qb, u1, dinv, g, lin_b, gam, beta, fcb):
    blk = 1000
    grid = N // blk
    return pl.pallas_call(
        _post_body,
        grid=(grid,),
        in_specs=[pl.BlockSpec((blk, W128), lambda i: (i, 0))] * 3
        + [
            pl.BlockSpec((blk, 1), lambda i: (i, 0)),
            pl.BlockSpec((OUT, blk, HID), lambda i: (0, i, 0)),
            pl.BlockSpec((1, HID), lambda i: (0, 0)),
            pl.BlockSpec((1, 1), lambda i: (0, 0)),
            pl.BlockSpec((1, 1), lambda i: (0, 0)),
            pl.BlockSpec((1, OUT), lambda i: (0, 0)),
        ],
        out_specs=pl.BlockSpec((1, OUT), lambda i: (0, 0)),
        out_shape=jax.ShapeDtypeStruct((1, OUT), jnp.float32),
    )(qa, qb, u1, dinv, g, lin_b, gam, beta, fcb)


@functools.partial(
    pl.kernel,
    out_type=jax.ShapeDtypeStruct((2, NPAD), jnp.float32),
    mesh=_SC_MESH,
    scratch_types=[
        pltpu.VMEM_SHARED((NPAD,), jnp.float32),
        pltpu.VMEM((CHUNK,), jnp.int32),
        pltpu.VMEM((CHUNK,), jnp.float32),
        pltpu.VMEM((ROWS_PT,), jnp.float32),
    ],
)
def _sc_deg(dst_hbm, w_hbm, out_hbm, acc, dbuf, wbuf, zbuf):
    cid = lax.axis_index("c")
    sid = lax.axis_index("s")
    wid = cid * NS + sid
    rb = sid * ROWS_PT

    def zf(i, _):
        zbuf[pl.ds(i * 16, 16)] = jnp.zeros((16,), jnp.float32)
        return 0

    lax.fori_loop(0, ROWS_PT // 16, zf, 0)
    pltpu.sync_copy(zbuf, acc.at[pl.ds(rb, ROWS_PT)])
    plsc.subcore_barrier()

    def body(c, _):
        pltpu.sync_copy(dst_hbm.at[wid, c], dbuf)
        pltpu.sync_copy(w_hbm.at[wid, c], wbuf)
        pltpu.sync_copy(wbuf, acc.at[dbuf], add=True)
        return 0

    lax.fori_loop(0, NCHUNK, body, 0)
    plsc.subcore_barrier()
    pltpu.sync_copy(acc.at[pl.ds(rb, ROWS_PT)],
                    out_hbm.at[cid, pl.ds(rb, ROWS_PT)])


@functools.partial(
    pl.kernel,
    out_type=jax.ShapeDtypeStruct((2, NPAD, W128), jnp.float32),
    mesh=_SC_MESH,
    scratch_types=[
        pltpu.VMEM_SHARED((NPAD, W128), jnp.float32),
        pltpu.VMEM((CHUNK,), jnp.int32),
        pltpu.VMEM((CHUNK,), jnp.int32),
        pltpu.VMEM((CHUNK,), jnp.float32),
        pltpu.VMEM((CHUNK, W128), jnp.float32),
        pltpu.SemaphoreType.DMA,
    ],
)
def _sc_prop(u_hbm, src_hbm, dst_hbm, w_hbm, out_hbm,
             acc, sbuf, dbuf, wbuf, rbuf, sem):
    cid = lax.axis_index("c")
    sid = lax.axis_index("s")
    wid = cid * NS + sid
    rb = sid * ROWS_PT
    pltpu.sync_copy(u_hbm.at[pl.ds(rb, ROWS_PT)], acc.at[pl.ds(rb, ROWS_PT)])
    plsc.subcore_barrier()

    dnums = lax.GatherDimensionNumbers(
        offset_dims=(), collapsed_slice_dims=(0,), start_index_map=(0,))

    def body(c, _):
        pltpu.sync_copy(src_hbm.at[wid, c], sbuf)
        pltpu.sync_copy(dst_hbm.at[wid, c], dbuf)
        pltpu.sync_copy(w_hbm.at[wid, c], wbuf)
        pltpu.async_copy(u_hbm.at[sbuf], rbuf, sem).wait()

        def gbody(g, _):
            wv = wbuf[pl.ds(g * 16, 16)]
            for k in range(16):
                ws = lax.gather(wv, jnp.full((16, 1), k, jnp.int32), dnums,
                                (1,),
                                mode=lax.GatherScatterMode.PROMISE_IN_BOUNDS)
                e = g * 16 + k
                for j in range(HID // 16):
                    rbuf[e, pl.ds(16 * j, 16)] = rbuf[e, pl.ds(16 * j, 16)] * ws
            return 0

        lax.fori_loop(0, CHUNK // 16, gbody, 0)
        pltpu.sync_copy(rbuf, acc.at[dbuf], add=True)
        return 0

    lax.fori_loop(0, NCHUNK, body, 0)
    plsc.subcore_barrier()
    pltpu.sync_copy(acc.at[pl.ds(rb, ROWS_PT)],
                    out_hbm.at[cid, pl.ds(rb, ROWS_PT)])


def kernel(x, edge_index, edge_weights, lin_w, lin_b, bn_gamma, bn_beta,
           fc_w, fc_b):
    xp = jnp.pad(x[0], ((0, NPAD - N), (0, 0)))
    npad_e = EPAD - E
    src = jnp.pad(edge_index[0], (0, npad_e), constant_values=TRASH)
    dst = jnp.pad(edge_index[1], (0, npad_e), constant_values=TRASH)
    w = jnp.pad(edge_weights, (0, npad_e))
    src_pk = src.reshape(NTILES, NCHUNK, CHUNK)
    dst_pk = dst.reshape(NTILES, NCHUNK, CHUNK)
    w_pk = w.reshape(NTILES, NCHUNK, CHUNK)
    g = jnp.repeat((fc_w * (1.0 / POOL)).reshape(OUT, N, POOL), POOL, axis=2)

    degp = _sc_deg(dst_pk, w_pk)
    u0, dinv = _tc_pre(xp, lin_w, degp[0][:, None], degp[1][:, None])
    p = _sc_prop(u0, src_pk, dst_pk, w_pk)
    u1 = _tc_mid(p[0], p[1], u0, dinv)
    q = _sc_prop(u1, src_pk, dst_pk, w_pk)
    return _tc_post(q[0], q[1], u1, dinv, g, lin_b[None, :], bn_gamma[None, :],
                    bn_beta[None, :], fc_b[None, :])
